# baseline (speedup 1.0000x reference)
import functools

import jax
import jax.numpy as jnp
from jax import lax
from jax.experimental import pallas as pl
from jax.experimental.pallas import tpu as pltpu
from jax.experimental.pallas import tpu_sc as plsc

EMBED_DIM = 16
BATCH = 16384
TCOL = 128
GROUP = 16
STAGE_COLS = GROUP * TCOL
NBUF = 3

B_TC = 4096
B_SC = BATCH - B_TC
SC_ROWS = B_SC // (32 * TCOL)
NGROUP = B_SC // (32 * GROUP)
TC_PER_STEP = 8


def _sc_kernel(uid_hbm, aid_hbm, ut_hbm, at_hbm, wb_hbm, out_hbm,
               uidx_v, aidx_v, buf0, buf1, buf2, out_v, wb_v,
               sem0, sem1, sem2):
    nc = lax.axis_size("c")
    wid = lax.axis_index("s") * nc + lax.axis_index("c")

    pltpu.sync_copy(uid_hbm.at[wid], uidx_v)
    pltpu.sync_copy(aid_hbm.at[wid], aidx_v)
    pltpu.sync_copy(wb_hbm, wb_v)

    bufs = (buf0, buf1, buf2)
    sems = (sem0, sem1, sem2)
    lanes = lax.iota(jnp.int32, 16)
    wvec = wb_v[pl.ds(0, EMBED_DIM)]
    wvec_a = wb_v[pl.ds(EMBED_DIM, EMBED_DIM)]
    bias = wb_v[pl.ds(2 * EMBED_DIM, EMBED_DIM)][0]

    def run_pass(idx_ref, tbl, ws, first):
        def load_ids(g):
            return idx_ref[g >> 3 if not isinstance(g, int) else g // 8,
                           pl.ds((g & 7 if not isinstance(g, int) else g % 8) * 16, 16)]

        def fire(g, par):
            idv = load_ids(g)
            for j in range(GROUP):
                cs = (idv[j] >> 7) << 7
                cs = pl.multiple_of(cs, TCOL)
                pltpu.async_copy(tbl.at[:, pl.ds(cs, TCOL)],
                                 bufs[par].at[:, pl.ds(j * TCOL, TCOL)],
                                 sems[par])

        def drain(par):
            pltpu.make_async_copy(tbl.at[:, pl.ds(0, STAGE_COLS)],
                                  bufs[par], sems[par]).wait()

        def compute(g, par):
            idv = load_ids(g)
            colv = lanes * TCOL + (idv & (TCOL - 1))
            sl = pl.ds(g * 16, 16)
            if first:
                acc = jnp.zeros((16,), jnp.float32) + bias
            else:
                acc = out_v[sl]
            for d in range(EMBED_DIM):
                vals = plsc.load_gather(bufs[par], [jnp.full((16,), d, jnp.int32),
                                                    colv])
                acc = acc + vals * ws[d]
            out_v[sl] = acc

        for p in range(NBUF):
            fire(p, p)

        def body(k, _):
            for p in range(NBUF):
                g = NBUF * k + p
                drain(p)
                compute(g, p)

                @pl.when(g + NBUF < NGROUP)
                def _():
                    fire(g + NBUF, p)

            return 0

        lax.fori_loop(0, NGROUP // NBUF, body, 0)
        for g in range(NBUF * (NGROUP // NBUF), NGROUP):
            p = g % NBUF
            drain(p)
            compute(g, p)

    wus = [wvec[d] for d in range(EMBED_DIM)]
    was = [wvec_a[d] for d in range(EMBED_DIM)]
    run_pass(uidx_v, ut_hbm, wus, True)
    run_pass(aidx_v, at_hbm, was, False)

    bpw = NGROUP * GROUP
    pltpu.sync_copy(out_v, out_hbm.at[pl.ds(wid * bpw, bpw)])


def _tc_kernel(uids_ref, aids_ref, *refs):
    ublks = refs[:TC_PER_STEP]
    ablks = refs[TC_PER_STEP:2 * TC_PER_STEP]
    w2, b2, out_ref = refs[2 * TC_PER_STEP:]
    i = pl.program_id(0)
    lane = jax.lax.broadcasted_iota(jnp.int32, (EMBED_DIM, TCOL), 1)
    wu = w2[:, 0:1]
    wa = w2[:, 1:2]
    parts = []
    for j in range(TC_PER_STEP):
        uoff = uids_ref[i * TC_PER_STEP + j] & (TCOL - 1)
        aoff = aids_ref[i * TC_PER_STEP + j] & (TCOL - 1)
        uval = jnp.sum(jnp.where(lane == uoff, ublks[j][...], 0.0) * wu)
        aval = jnp.sum(jnp.where(lane == aoff, ablks[j][...], 0.0) * wa)
        parts.append((uval + aval).reshape(1, 1))
    row = jnp.concatenate(parts, axis=1) + b2[...]
    rowsel = jax.lax.broadcasted_iota(
        jnp.int32, (TC_PER_STEP, TC_PER_STEP), 0) == (i & (TC_PER_STEP - 1))
    out_ref[...] = jnp.where(rowsel, jnp.broadcast_to(row, rowsel.shape),
                             out_ref[...])


def _tc_gather(uids, aids, ut_t, at_t, w2, b2):
    nsteps = B_TC // TC_PER_STEP
    tbl_spec = lambda ids_idx: [
        pl.BlockSpec(
            (EMBED_DIM, TCOL),
            functools.partial(
                lambda j, i, u, a: (0, (u[i * TC_PER_STEP + j]
                                        if ids_idx == 0 else
                                        a[i * TC_PER_STEP + j]) >> 7), j))
        for j in range(TC_PER_STEP)
    ]
    grid_spec = pltpu.PrefetchScalarGridSpec(
        num_scalar_prefetch=2,
        grid=(nsteps,),
        in_specs=[
            *tbl_spec(0),
            *tbl_spec(1),
            pl.BlockSpec((EMBED_DIM, 2), lambda i, u, a: (0, 0)),
            pl.BlockSpec((1, 1), lambda i, u, a: (0, 0)),
        ],
        out_specs=pl.BlockSpec((TC_PER_STEP, TC_PER_STEP),
                               lambda i, u, a: (i // TC_PER_STEP, 0)),
    )
    out = pl.pallas_call(
        _tc_kernel,
        grid_spec=grid_spec,
        out_shape=jax.ShapeDtypeStruct((nsteps, TC_PER_STEP), jnp.float32),
    )(uids, aids, *([ut_t] * TC_PER_STEP), *([at_t] * TC_PER_STEP), w2, b2)
    return out.reshape(B_TC)


def kernel(user_ids, article_ids, user_table, article_table, fc_w, fc_b):
    info = plsc.get_sparse_core_info()
    nw = info.num_cores * info.num_subcores
    assert B_SC == nw * NGROUP * GROUP

    uids = user_ids.astype(jnp.int32)
    aids = article_ids.astype(jnp.int32)
    uid2 = uids[:B_SC].reshape(32, SC_ROWS, TCOL)
    aid2 = aids[:B_SC].reshape(32, SC_ROWS, TCOL)
    ut_t = user_table.T
    at_t = article_table.T
    wb = jnp.concatenate([fc_w.reshape(-1), fc_b.reshape(-1),
                          jnp.zeros((15,), jnp.float32)])

    mesh = plsc.VectorSubcoreMesh(core_axis_name="c", subcore_axis_name="s")
    sc_out = pl.kernel(
        _sc_kernel,
        mesh=mesh,
        compiler_params=pltpu.CompilerParams(needs_layout_passes=False),
        out_type=jax.ShapeDtypeStruct((B_SC,), jnp.float32),
        scratch_types=[
            pltpu.VMEM((SC_ROWS, TCOL), jnp.int32),
            pltpu.VMEM((SC_ROWS, TCOL), jnp.int32),
            pltpu.VMEM((EMBED_DIM, STAGE_COLS), jnp.float32),
            pltpu.VMEM((EMBED_DIM, STAGE_COLS), jnp.float32),
            pltpu.VMEM((EMBED_DIM, STAGE_COLS), jnp.float32),
            pltpu.VMEM((NGROUP * GROUP,), jnp.float32),
            pltpu.VMEM((48,), jnp.float32),
            pltpu.SemaphoreType.DMA,
            pltpu.SemaphoreType.DMA,
            pltpu.SemaphoreType.DMA,
        ],
    )(uid2, aid2, ut_t, at_t, wb)

    w2 = fc_w.reshape(2, EMBED_DIM).T
    b2 = fc_b.reshape(1, 1)
    tc_out = _tc_gather(uids[B_SC:], aids[B_SC:], ut_t, at_t, w2, b2)

    return jnp.concatenate([sc_out, tc_out]).reshape(BATCH, 1)


# speedup vs baseline: 3.0875x; 3.0875x over previous
import jax
import jax.numpy as jnp
from jax import lax
from jax.experimental import pallas as pl
from jax.experimental.pallas import tpu as pltpu
from jax.experimental.pallas import tpu_sc as plsc

EMBED_DIM = 16
BATCH = 16384
GROUP = 16
NGROUP = 32
TCOL = 128
STAGE_COLS = GROUP * TCOL


NBUF = 3


def _sc_kernel(uid_hbm, aid_hbm, ut_hbm, at_hbm, wb_hbm, out_hbm,
               uidx_v, aidx_v, buf0, buf1, buf2, out_v, wb_v,
               sem0, sem1, sem2):
    nc = lax.axis_size("c")
    wid = lax.axis_index("s") * nc + lax.axis_index("c")

    pltpu.sync_copy(uid_hbm.at[pl.ds(wid * 4, 4)], uidx_v)
    pltpu.sync_copy(aid_hbm.at[pl.ds(wid * 4, 4)], aidx_v)
    pltpu.sync_copy(wb_hbm, wb_v)

    bufs = (buf0, buf1, buf2)
    sems = (sem0, sem1, sem2)
    lanes = lax.iota(jnp.int32, 16)
    wvec = wb_v[pl.ds(0, EMBED_DIM)]
    wvec_a = wb_v[pl.ds(EMBED_DIM, EMBED_DIM)]
    bias = wb_v[pl.ds(2 * EMBED_DIM, EMBED_DIM)][0]

    def run_pass(idx_ref, tbl, ws, first):
        def load_ids(g):
            return idx_ref[g // 8 if isinstance(g, int) else g >> 3,
                           pl.ds((g % 8 if isinstance(g, int) else g & 7) * 16, 16)]

        def fire(g, par):
            idv = load_ids(g)
            for j in range(GROUP):
                cs = (idv[j] >> 7) << 7
                cs = pl.multiple_of(cs, TCOL)
                pltpu.async_copy(tbl.at[:, pl.ds(cs, TCOL)],
                                 bufs[par].at[:, pl.ds(j * TCOL, TCOL)],
                                 sems[par])

        def drain(par):
            pltpu.make_async_copy(tbl.at[:, pl.ds(0, STAGE_COLS)],
                                  bufs[par], sems[par]).wait()

        def compute(g, par):
            idv = load_ids(g)
            colv = lanes * TCOL + (idv & (TCOL - 1))
            sl = pl.ds(g * 16, 16)
            if first:
                acc = jnp.zeros((16,), jnp.float32) + bias
            else:
                acc = out_v[sl]
            for d in range(EMBED_DIM):
                vals = plsc.load_gather(bufs[par], [jnp.full((16,), d, jnp.int32),
                                                    colv])
                acc = acc + vals * ws[d]
            out_v[sl] = acc

        for p in range(NBUF):
            fire(p, p)

        def body(k, _):
            for p in range(NBUF):
                g = NBUF * k + p
                drain(p)
                compute(g, p)

                @pl.when(g + NBUF < NGROUP)
                def _():
                    fire(g + NBUF, p)

            return 0

        lax.fori_loop(0, NGROUP // NBUF, body, 0)
        for g in range(NBUF * (NGROUP // NBUF), NGROUP):
            p = g % NBUF
            drain(p)
            compute(g, p)

    wus = [wvec[d] for d in range(EMBED_DIM)]
    was = [wvec_a[d] for d in range(EMBED_DIM)]
    run_pass(uidx_v, ut_hbm, wus, True)
    run_pass(aidx_v, at_hbm, was, False)

    pltpu.sync_copy(out_v, out_hbm.at[pl.ds(wid * 512, 512)])


def kernel(user_ids, article_ids, user_table, article_table, fc_w, fc_b):
    info = plsc.get_sparse_core_info()
    nw = info.num_cores * info.num_subcores
    assert BATCH == nw * 512

    uid2 = user_ids.reshape(-1, 128).astype(jnp.int32)
    aid2 = article_ids.reshape(-1, 128).astype(jnp.int32)
    ut_t = user_table.T
    at_t = article_table.T
    wb = jnp.concatenate([fc_w.reshape(-1), fc_b.reshape(-1),
                          jnp.zeros((15,), jnp.float32)])

    mesh = plsc.VectorSubcoreMesh(core_axis_name="c", subcore_axis_name="s")
    out = pl.kernel(
        _sc_kernel,
        mesh=mesh,
        compiler_params=pltpu.CompilerParams(needs_layout_passes=False),
        out_type=jax.ShapeDtypeStruct((BATCH,), jnp.float32),
        scratch_types=[
            pltpu.VMEM((4, 128), jnp.int32),
            pltpu.VMEM((4, 128), jnp.int32),
            pltpu.VMEM((EMBED_DIM, STAGE_COLS), jnp.float32),
            pltpu.VMEM((EMBED_DIM, STAGE_COLS), jnp.float32),
            pltpu.VMEM((EMBED_DIM, STAGE_COLS), jnp.float32),
            pltpu.VMEM((512,), jnp.float32),
            pltpu.VMEM((48,), jnp.float32),
            pltpu.SemaphoreType.DMA,
            pltpu.SemaphoreType.DMA,
            pltpu.SemaphoreType.DMA,
        ],
    )(uid2, aid2, ut_t, at_t, wb)
    return out.reshape(BATCH, 1)
